# baseline (speedup 1.0000x reference)
import functools

import jax
import jax.numpy as jnp
from jax import lax
from jax.experimental import pallas as pl
from jax.experimental.pallas import tpu as pltpu
from jax.experimental.pallas import tpu_sc as plsc

VOCAB_DIM = 100000
SEQ_LEN = 200
D_MODEL = 128
BATCH = 1024

NC = 2
NS = 16
NW = NC * NS
NCHUNK = BATCH // NW
HALF = SEQ_LEN // 2
NBUF = 4
LOOK = 2
IBLK = 4
NSUPER = NCHUNK // (2 * IBLK)


def _sc_body(data_hbm, tok_hbm, pos_hbm, out_hbm,
             idxa, idxb, rows0, rows1, rows2, rows3, pos_v,
             g0, g1, g2, g3, o0, o1, o2, o3, isem):
    wid = lax.axis_index("s") * NC + lax.axis_index("c")
    base = wid * NCHUNK
    idx_v = (idxa, idxb)
    rows_v = (rows0, rows1, rows2, rows3)
    gsem = (g0, g1, g2, g3)
    osem = (o0, o1, o2, o3)

    def load_block(k, buf):
        return pltpu.async_copy(data_hbm.at[pl.ds(base + IBLK * k, IBLK)],
                                idx_v[buf], isem)

    def wait_block(buf):
        pltpu.make_async_copy(data_hbm.at[pl.ds(base, IBLK)],
                              idx_v[buf], isem).wait()

    def gather_parts(c, ib, slot):
        iref = idx_v[ib]
        row = c % IBLK
        return ((tok_hbm.at[iref.at[row, 0]],
                 rows_v[slot].at[pl.ds(0, HALF)], gsem[slot]),
                (tok_hbm.at[iref.at[row, 1]],
                 rows_v[slot].at[pl.ds(HALF, HALF)], gsem[slot]))

    def issue_gather(c, ib, slot):
        for src, dst, sem in gather_parts(c, ib, slot):
            pltpu.async_copy(src, dst, sem)

    def wait_gather(c, ib, slot):
        for src, dst, sem in gather_parts(c, ib, slot):
            pltpu.make_async_copy(src, dst, sem).wait()

    def wait_out(slot):
        pltpu.make_async_copy(rows_v[slot], out_hbm.at[base], osem[slot]).wait()

    icp = load_block(0, 0)
    pcp = pltpu.async_copy(pos_hbm, pos_v, o0)
    icp.wait()
    issue_gather(0, 0, 0)
    issue_gather(1, 0, 1)
    pcp.wait()

    def super_group(g, carry):
        for j in range(2 * IBLK):
            c = 2 * IBLK * g + j
            slot = j % NBUF
            ns = (j + LOOK) % NBUF

            if j == 0:
                load_block(2 * g + 1, 1)
            if j == IBLK:
                @pl.when(g < NSUPER - 1)
                def _():
                    load_block(2 * g + 2, 0)
            if j == 2:
                wait_block(1)
            if j == 2 + IBLK:
                @pl.when(g < NSUPER - 1)
                def _():
                    wait_block(0)

            if j >= LOOK:
                wait_out(ns)
            else:
                @pl.when(g >= 1)
                def _():
                    wait_out(ns)

            nib = ((j + LOOK) // IBLK) % 2
            if j < 2 * IBLK - LOOK:
                issue_gather(c + LOOK, nib, ns)
            else:
                @pl.when(g < NSUPER - 1)
                def _():
                    issue_gather(c + LOOK, nib, ns)

            wait_gather(c, (j // IBLK) % 2, slot)

            @plsc.parallel_loop(0, SEQ_LEN, step=1, unroll=5)
            def addrow(i):
                for jj in range(D_MODEL // 16):
                    sl = pl.ds(jj * 16, 16)
                    rows_v[slot][i, sl] = rows_v[slot][i, sl] + pos_v[i, sl]

            pltpu.async_copy(rows_v[slot], out_hbm.at[base + c], osem[slot])
        return carry

    lax.fori_loop(0, NSUPER, super_group, 0)
    for k in range(LOOK):
        wait_out((NCHUNK - LOOK + k) % NBUF)


def kernel(data, token_table, pos_table):
    data3 = data.reshape(BATCH, 2, HALF).astype(jnp.int32)
    mesh = plsc.VectorSubcoreMesh(core_axis_name="c", subcore_axis_name="s")
    run = functools.partial(
        pl.kernel,
        out_type=jax.ShapeDtypeStruct((BATCH, SEQ_LEN, D_MODEL), jnp.float32),
        mesh=mesh,
        scratch_types=[
            pltpu.VMEM((IBLK, 2, HALF), jnp.int32),
            pltpu.VMEM((IBLK, 2, HALF), jnp.int32),
            pltpu.VMEM((SEQ_LEN, D_MODEL), jnp.float32),
            pltpu.VMEM((SEQ_LEN, D_MODEL), jnp.float32),
            pltpu.VMEM((SEQ_LEN, D_MODEL), jnp.float32),
            pltpu.VMEM((SEQ_LEN, D_MODEL), jnp.float32),
            pltpu.VMEM((SEQ_LEN, D_MODEL), jnp.float32),
            pltpu.SemaphoreType.DMA,
            pltpu.SemaphoreType.DMA,
            pltpu.SemaphoreType.DMA,
            pltpu.SemaphoreType.DMA,
            pltpu.SemaphoreType.DMA,
            pltpu.SemaphoreType.DMA,
            pltpu.SemaphoreType.DMA,
            pltpu.SemaphoreType.DMA,
            pltpu.SemaphoreType.DMA,
        ],
    )(_sc_body)
    return run(data3, token_table, pos_table)


# speedup vs baseline: 1.0129x; 1.0129x over previous
import functools

import jax
import jax.numpy as jnp
from jax import lax
from jax.experimental import pallas as pl
from jax.experimental.pallas import tpu as pltpu
from jax.experimental.pallas import tpu_sc as plsc

VOCAB_DIM = 100000
SEQ_LEN = 200
D_MODEL = 128
BATCH = 1024

NC = 2
NS = 16
NW = NC * NS
NCHUNK = BATCH // NW
HALF = SEQ_LEN // 2
NBUF = 3
NGROUP = NCHUNK // NBUF
NREM = NCHUNK - NGROUP * NBUF


def _sc_body(data_hbm, tok_hbm, pos_hbm, out_hbm,
             idx_all, rows0, rows1, rows2, pos_v, g0, g1, g2, o0, o1, o2):
    wid = lax.axis_index("s") * NC + lax.axis_index("c")
    base = wid * NCHUNK
    rows_v = (rows0, rows1, rows2)
    gsem = (g0, g1, g2)
    osem = (o0, o1, o2)

    icp = pltpu.async_copy(data_hbm.at[pl.ds(base, NCHUNK)], idx_all, g0)
    pcp = pltpu.async_copy(pos_hbm, pos_v, o0)
    icp.wait()

    def issue_gather(c, b):
        pltpu.async_copy(tok_hbm.at[idx_all.at[c, 0]],
                         rows_v[b].at[pl.ds(0, HALF)], gsem[b])
        pltpu.async_copy(tok_hbm.at[idx_all.at[c, 1]],
                         rows_v[b].at[pl.ds(HALF, HALF)], gsem[b])

    def wait_gather(c, b):
        pltpu.make_async_copy(tok_hbm.at[idx_all.at[c, 0]],
                              rows_v[b].at[pl.ds(0, HALF)], gsem[b]).wait()
        pltpu.make_async_copy(tok_hbm.at[idx_all.at[c, 1]],
                              rows_v[b].at[pl.ds(HALF, HALF)], gsem[b]).wait()

    def wait_out(b):
        pltpu.make_async_copy(rows_v[b], out_hbm.at[base], osem[b]).wait()

    def step(c, b):
        bn = (b + 1) % NBUF

        if isinstance(c, int):
            if c >= NBUF - 1:
                wait_out(bn)
            if c + 1 < NCHUNK:
                issue_gather(c + 1, bn)
        else:
            @pl.when(c >= NBUF - 1)
            def _():
                wait_out(bn)

            @pl.when(c + 1 < NCHUNK)
            def _():
                issue_gather(c + 1, bn)

        wait_gather(c, b)

        @plsc.parallel_loop(0, SEQ_LEN, step=1, unroll=5)
        def addrow(i):
            for j in range(D_MODEL // 16):
                sl = pl.ds(j * 16, 16)
                rows_v[b][i, sl] = rows_v[b][i, sl] + pos_v[i, sl]

        pltpu.async_copy(rows_v[b], out_hbm.at[base + c], osem[b])

    issue_gather(0, 0)
    pcp.wait()

    def group(g, carry):
        for b in range(NBUF):
            step(g * NBUF + b, b)
        return carry

    lax.fori_loop(0, NGROUP, group, 0)
    for k in range(NREM):
        step(NGROUP * NBUF + k, k)
    for k in range(NBUF - 1):
        wait_out((NCHUNK - (NBUF - 1) + k) % NBUF)


def kernel(data, token_table, pos_table):
    data3 = data.reshape(BATCH, 2, HALF).astype(jnp.int32)
    mesh = plsc.VectorSubcoreMesh(core_axis_name="c", subcore_axis_name="s")
    run = functools.partial(
        pl.kernel,
        out_type=jax.ShapeDtypeStruct((BATCH, SEQ_LEN, D_MODEL), jnp.float32),
        mesh=mesh,
        scratch_types=[
            pltpu.VMEM((NCHUNK, 2, HALF), jnp.int32),
            pltpu.VMEM((SEQ_LEN, D_MODEL), jnp.float32),
            pltpu.VMEM((SEQ_LEN, D_MODEL), jnp.float32),
            pltpu.VMEM((SEQ_LEN, D_MODEL), jnp.float32),
            pltpu.VMEM((SEQ_LEN, D_MODEL), jnp.float32),
            pltpu.SemaphoreType.DMA,
            pltpu.SemaphoreType.DMA,
            pltpu.SemaphoreType.DMA,
            pltpu.SemaphoreType.DMA,
            pltpu.SemaphoreType.DMA,
            pltpu.SemaphoreType.DMA,
        ],
    )(_sc_body)
    return run(data3, token_table, pos_table)
